# baseline (speedup 1.0000x reference)
import jax
import jax.numpy as jnp
from jax import lax
from jax.experimental import pallas as pl
from jax.experimental.pallas import tpu as pltpu
from jax.experimental.pallas import tpu_sc as plsc

VOCAB = 100000
MAX_POS = 2048
DIM = 768
B = 4
S = 2048
N = B * S

_info = plsc.get_sparse_core_info()
NC, NS, L = _info.num_cores, _info.num_subcores, _info.num_lanes
NW = NC * NS
PPW = S // NW
TOK_PER_W = B * PPW
PCH = 16
ROWS = B * PCH
NCHUNK = PPW // PCH
DV = DIM // L
NBUF = 2
AHEAD = 1


def _body(ids_hbm, wte_hbm, wpe_hbm, out_hbm, idx_v, *refs):
    rows = refs[:NBUF]
    wpes = refs[NBUF:2 * NBUF]
    gsem = refs[2 * NBUF:3 * NBUF]
    ssem = refs[3 * NBUF:4 * NBUF]
    wsem = refs[4 * NBUF:5 * NBUF]
    wid = lax.axis_index("s") * NC + lax.axis_index("c")
    pos0 = wid * PPW
    pltpu.sync_copy(ids_hbm.at[pl.ds(wid * TOK_PER_W, TOK_PER_W)], idx_v)

    def start(c):
        buf = c % NBUF
        g = pltpu.async_copy(
            wte_hbm.at[idx_v.at[pl.ds(c * ROWS, ROWS)]], rows[buf], gsem[buf])
        w = pltpu.async_copy(
            wpe_hbm.at[pl.ds(pos0 + c * PCH, PCH)], wpes[buf], wsem[buf])
        return g, w

    gathers = [None] * NCHUNK
    stores = [[None] * B for _ in range(NCHUNK)]
    store_waited = [False] * NCHUNK
    for c in range(AHEAD):
        gathers[c] = start(c)
    for c in range(NCHUNK):
        buf = c % NBUF
        gathers[c][0].wait()
        gathers[c][1].wait()

        @pl.loop(0, PCH)
        def _(p):
            for j in range(DV):
                sl = pl.ds(j * L, L)
                w = wpes[buf][p, sl]
                for b in range(B):
                    plsc.addupdate(rows[buf].at[b * PCH + p, sl], w)

        for b in range(B):
            stores[c][b] = pltpu.async_copy(
                rows[buf].at[pl.ds(b * PCH, PCH)],
                out_hbm.at[pl.ds(b * S + pos0 + c * PCH, PCH)], ssem[buf])
        if c + AHEAD < NCHUNK:
            prev = c + AHEAD - NBUF
            if prev >= 0:
                for b in range(B):
                    stores[prev][b].wait()
                store_waited[prev] = True
            gathers[c + AHEAD] = start(c + AHEAD)
    for c in range(NCHUNK):
        if not store_waited[c]:
            for b in range(B):
                stores[c][b].wait()


@jax.jit
def _run(ids_flat, wte, wpe):
    mesh = plsc.VectorSubcoreMesh(core_axis_name="c", subcore_axis_name="s")
    return pl.kernel(
        _body,
        mesh=mesh,
        out_type=jax.ShapeDtypeStruct((N, DIM), jnp.float32),
        scratch_types=(
            [pltpu.VMEM((TOK_PER_W,), jnp.int32)]
            + [pltpu.VMEM((ROWS, DIM), jnp.float32) for _ in range(NBUF)]
            + [pltpu.VMEM((PCH, DIM), jnp.float32) for _ in range(NBUF)]
            + [pltpu.SemaphoreType.DMA for _ in range(3 * NBUF)]
        ),
    )(ids_flat, wte, wpe)


def kernel(input_ids, wte, wpe):
    ids_r = (input_ids.astype(jnp.int32)
             .reshape(B, NW, NCHUNK, PCH)
             .transpose(1, 2, 0, 3)
             .reshape(N))
    out = _run(ids_r, wte, wpe)
    return out.reshape(B, S, DIM)


# speedup vs baseline: 1.1470x; 1.1470x over previous
import jax
import jax.numpy as jnp
from jax import lax
from jax.experimental import pallas as pl
from jax.experimental.pallas import tpu as pltpu
from jax.experimental.pallas import tpu_sc as plsc

VOCAB = 100000
MAX_POS = 2048
DIM = 768
B = 4
S = 2048
N = B * S

_info = plsc.get_sparse_core_info()
NC, NS, L = _info.num_cores, _info.num_subcores, _info.num_lanes
NW = NC * NS
PPW = S // NW
TOK_PER_W = B * PPW
PCH = 8
ROWS = B * PCH
NCHUNK = PPW // PCH
DV = DIM // L
NBUF = 4
AHEAD = 3


def _body(ids_hbm, wte_hbm, wpe_hbm, out_hbm, idx_v, *refs):
    rows = refs[:NBUF]
    wpes = refs[NBUF:2 * NBUF]
    gsem = refs[2 * NBUF:3 * NBUF]
    ssem = refs[3 * NBUF:4 * NBUF]
    wsem = refs[4 * NBUF:5 * NBUF]
    wid = lax.axis_index("s") * NC + lax.axis_index("c")
    pos0 = wid * PPW
    pltpu.sync_copy(ids_hbm.at[pl.ds(wid * TOK_PER_W, TOK_PER_W)], idx_v)

    def start(c):
        buf = c % NBUF
        g = pltpu.async_copy(
            wte_hbm.at[idx_v.at[pl.ds(c * ROWS, ROWS)]], rows[buf], gsem[buf])
        w = pltpu.async_copy(
            wpe_hbm.at[pl.ds(pos0 + c * PCH, PCH)], wpes[buf], wsem[buf])
        return g, w

    gathers = [None] * NCHUNK
    stores = [[None] * B for _ in range(NCHUNK)]
    store_waited = [False] * NCHUNK
    for c in range(AHEAD):
        gathers[c] = start(c)
    for c in range(NCHUNK):
        buf = c % NBUF
        gathers[c][0].wait()
        gathers[c][1].wait()

        @pl.loop(0, PCH)
        def _(p):
            for j in range(DV):
                sl = pl.ds(j * L, L)
                w = wpes[buf][p, sl]
                for b in range(B):
                    plsc.addupdate(rows[buf].at[b * PCH + p, sl], w)

        for b in range(B):
            stores[c][b] = pltpu.async_copy(
                rows[buf].at[pl.ds(b * PCH, PCH)],
                out_hbm.at[pl.ds(b * S + pos0 + c * PCH, PCH)], ssem[buf])
        if c + AHEAD < NCHUNK:
            prev = c + AHEAD - NBUF
            if prev >= 0:
                for b in range(B):
                    stores[prev][b].wait()
                store_waited[prev] = True
            gathers[c + AHEAD] = start(c + AHEAD)
    for c in range(NCHUNK):
        if not store_waited[c]:
            for b in range(B):
                stores[c][b].wait()


@jax.jit
def _run(ids_flat, wte, wpe):
    mesh = plsc.VectorSubcoreMesh(core_axis_name="c", subcore_axis_name="s")
    return pl.kernel(
        _body,
        mesh=mesh,
        out_type=jax.ShapeDtypeStruct((N, DIM), jnp.float32),
        scratch_types=(
            [pltpu.VMEM((TOK_PER_W,), jnp.int32)]
            + [pltpu.VMEM((ROWS, DIM), jnp.float32) for _ in range(NBUF)]
            + [pltpu.VMEM((PCH, DIM), jnp.float32) for _ in range(NBUF)]
            + [pltpu.SemaphoreType.DMA for _ in range(3 * NBUF)]
        ),
    )(ids_flat, wte, wpe)


def kernel(input_ids, wte, wpe):
    ids_r = (input_ids.astype(jnp.int32)
             .reshape(B, NW, NCHUNK, PCH)
             .transpose(1, 2, 0, 3)
             .reshape(N))
    out = _run(ids_r, wte, wpe)
    return out.reshape(B, S, DIM)
